# Initial kernel scaffold; baseline (speedup 1.0000x reference)
import jax
import jax.numpy as jnp
from jax.experimental import pallas as pl


def kernel(continuous, binary, categorical, W_wide, b_wide, emb_tables, W1, b1, g1, be1, W2, b2, g2, be2, W3, b3):
    raise NotImplementedError("write your pallas kernel here")


# speedup vs baseline: 7.5387x; 7.5387x over previous
import numpy as np
import jax
import jax.numpy as jnp
from jax import lax
from jax.experimental import pallas as pl
from jax.experimental.pallas import tpu as pltpu, tpu_sc as plsc

_B = 16384
_N_CONT = 13
_N_BIN = 13
_N_CAT = 26
_VOCAB = 100000
_EMB = 16
_EPS = 1e-5

_NC = 2
_NS = 16
_NW = _NC * _NS
_CHUNK = 128
_GROUP = 8
_GROUP_ROWS = _CHUNK * _GROUP


def _sc_gather(tables_flat, idx2d):
    n_chunks = idx2d.shape[0]
    nrows = n_chunks * _CHUNK
    chunks_per_w = n_chunks // _NW
    groups = chunks_per_w // _GROUP

    mesh = plsc.VectorSubcoreMesh(core_axis_name="c", subcore_axis_name="s")

    def body(tables_hbm, idx_hbm, out_hbm, idx_v, rows_v, sem):
        wid = lax.axis_index("s") * _NC + lax.axis_index("c")
        crow = wid * chunks_per_w
        pltpu.sync_copy(idx_hbm.at[pl.ds(crow, chunks_per_w)], idx_v)

        def group(g, carry):
            descs = []
            for s in range(_GROUP):
                j = g * _GROUP + s
                d = pltpu.async_copy(
                    tables_hbm.at[idx_v.at[j]],
                    rows_v.at[pl.ds(s * _CHUNK, _CHUNK)],
                    sem,
                )
                descs.append(d)
            for d in descs:
                d.wait()
            pltpu.sync_copy(
                rows_v,
                out_hbm.at[pl.ds((crow + g * _GROUP) * _CHUNK, _GROUP_ROWS)],
            )
            return carry

        lax.fori_loop(0, groups, group, 0)

    return pl.kernel(
        body,
        out_type=jax.ShapeDtypeStruct((nrows, _EMB), jnp.float32),
        mesh=mesh,
        scratch_types=[
            pltpu.VMEM((chunks_per_w, _CHUNK), jnp.int32),
            pltpu.VMEM((_GROUP_ROWS, _EMB), jnp.float32),
            pltpu.SemaphoreType.DMA,
        ],
        compiler_params=pltpu.CompilerParams(use_tc_tiling_on_sc=False),
    )(tables_flat, idx2d)


def _tc_mlp(xw, emb, Ww, W1, W2, W3, b1r, g1r, be1r, b2r, g2r, be2r, c0):
    B, deep_emb = emb.shape
    R = 2048
    inv = float(1.0 / np.sqrt(1.0 + _EPS))

    def body(xw_ref, emb_ref, ww_ref, w1_ref, w2_ref, w3_ref,
             b1_ref, g1_ref, be1_ref, b2_ref, g2_ref, be2_ref, c0_ref,
             out_ref):
        xw_blk = xw_ref[...]
        emb_blk = emb_ref[...]
        dn = (((1,), (1,)), ((), ()))
        hi = jax.lax.Precision.HIGHEST
        wide = lax.dot_general(xw_blk, ww_ref[...], dn, precision=hi,
                               preferred_element_type=jnp.float32)
        w1 = w1_ref[...]
        h = lax.dot_general(xw_blk[:, :26], w1[:, :26], dn, precision=hi,
                            preferred_element_type=jnp.float32)
        h = h + lax.dot_general(emb_blk, w1[:, 26:], dn, precision=hi,
                                preferred_element_type=jnp.float32)
        h = (h + b1_ref[...]) * (g1_ref[...] * inv) + be1_ref[...]
        h = jnp.maximum(h, 0.0)
        h = lax.dot_general(h, w2_ref[...], dn, precision=hi,
                            preferred_element_type=jnp.float32)
        h = (h + b2_ref[...]) * (g2_ref[...] * inv) + be2_ref[...]
        h = jnp.maximum(h, 0.0)
        deep = lax.dot_general(h, w3_ref[...], dn, precision=hi,
                               preferred_element_type=jnp.float32)
        out_ref[...] = 0.5 * wide + 0.5 * deep + c0_ref[...]

    full = lambda shape: pl.BlockSpec(shape, lambda i: (0,) * len(shape))
    return pl.pallas_call(
        body,
        grid=(B // R,),
        in_specs=[
            pl.BlockSpec((R, xw.shape[1]), lambda i: (i, 0)),
            pl.BlockSpec((R, deep_emb), lambda i: (i, 0)),
            full(Ww.shape),
            full(W1.shape),
            full(W2.shape),
            full(W3.shape),
            full(b1r.shape),
            full(g1r.shape),
            full(be1r.shape),
            full(b2r.shape),
            full(g2r.shape),
            full(be2r.shape),
            full(c0.shape),
        ],
        out_specs=pl.BlockSpec((R, 1), lambda i: (i, 0)),
        out_shape=jax.ShapeDtypeStruct((B, 1), jnp.float32),
    )(xw, emb, Ww, W1, W2, W3, b1r, g1r, be1r, b2r, g2r, be2r, c0)


def kernel(continuous, binary, categorical, W_wide, b_wide, emb_tables,
           W1, b1, g1, be1, W2, b2, g2, be2, W3, b3):
    B = continuous.shape[0]
    catf = categorical.astype(jnp.float32)
    xw = jnp.concatenate([continuous, binary, catf], axis=1)

    cat32 = categorical.astype(jnp.int32)
    offs = jnp.arange(_N_CAT, dtype=jnp.int32) * _VOCAB
    idx2d = (cat32 + offs[None, :]).reshape(-1, _CHUNK)

    tables_flat = emb_tables.reshape(_N_CAT * _VOCAB, _EMB)
    gathered = _sc_gather(tables_flat, idx2d)
    emb_flat = gathered.reshape(B, _N_CAT * _EMB)

    c0 = (0.5 * (b_wide + b3)).reshape(1, 1)
    out2d = _tc_mlp(
        xw, emb_flat, W_wide, W1, W2, W3,
        b1.reshape(1, -1), g1.reshape(1, -1), be1.reshape(1, -1),
        b2.reshape(1, -1), g2.reshape(1, -1), be2.reshape(1, -1),
        c0,
    )
    return out2d.reshape(B)
